# baseline (speedup 1.0000x reference)
import functools

import jax
import jax.numpy as jnp
from jax import lax
from jax.experimental import pallas as pl
from jax.experimental.pallas import tpu as pltpu

_BN_EPS = 1e-5


def _sigmoid(x):
    return 1.0 / (1.0 + jnp.exp(-x))


def _conv_block_body(x_ref, *refs, nb, H, T, couts, W):
    ncv = len(couts)
    w_refs = [refs[2 * i] for i in range(ncv)]
    b_refs = [refs[2 * i + 1] for i in range(ncv)]
    m_ref = refs[2 * ncv]
    idx_ref = refs[2 * ncv + 1]

    cur = x_ref[...]
    M = nb * H
    for i in range(ncv):
        hp = jnp.pad(cur, ((0, 0), (1, 1), (0, 0)))
        lhs = jnp.concatenate([hp[:, ki:ki + H, :] for ki in range(3)],
                              axis=-1).reshape(M, -1)
        y = jnp.dot(lhs, w_refs[i][...], preferred_element_type=jnp.float32)
        y = jnp.maximum(y + b_refs[i][...], 0.0)
        if i < ncv - 1:
            cur = y.reshape(nb, H, y.shape[-1])

    C = couts[-1]
    H2 = H // 2
    WC = W * C
    ysh = jnp.pad(y, ((0, 0), (0, C)))[:, C:]
    mw = jnp.maximum(y, ysh)
    djr = jnp.where(y >= ysh, 0, 1).astype(jnp.int32)
    mh = mw.reshape(nb * H2, 2, WC)
    jh = djr.reshape(nb * H2, 2, WC)
    me, mo = mh[:, 0], mh[:, 1]
    sel = me >= mo
    m = jnp.where(sel, me, mo)
    di = jnp.where(sel, 0, 1).astype(jnp.int32)
    dj = jnp.where(sel, jh[:, 0], jh[:, 1])

    row = lax.broadcasted_iota(jnp.int32, (nb * H2, WC), 0)
    lane = lax.broadcasted_iota(jnp.int32, (nb * H2, WC), 1)
    n = pl.program_id(0) * nb + row // H2
    t = n % T
    ho = row % H2
    w = lane // C
    idx = t * (H * W) + (2 * ho + di) * W + (w + dj)

    m_ref[...] = m
    idx_ref[...] = idx


def _conv_block(xp, ws, bs, *, nb, H, T, couts):
    F = xp.shape[0]
    W = H
    C = couts[-1]
    H2 = H // 2
    in_specs = [pl.BlockSpec((nb, H, xp.shape[2]), lambda i: (i, 0, 0))]
    for w, b in zip(ws, bs):
        in_specs.append(pl.BlockSpec(w.shape, lambda i: (0, 0)))
        in_specs.append(pl.BlockSpec(b.shape, lambda i: (0, 0)))
    args = [xp]
    for w, b in zip(ws, bs):
        args.extend([w, b])
    out, idx = pl.pallas_call(
        functools.partial(_conv_block_body, nb=nb, H=H, T=T,
                          couts=couts, W=W),
        grid=(F // nb,),
        in_specs=in_specs,
        out_specs=[
            pl.BlockSpec((nb * H2, W * C), lambda i: (i, 0)),
            pl.BlockSpec((nb * H2, W * C), lambda i: (i, 0)),
        ],
        out_shape=[
            jax.ShapeDtypeStruct((F * H2, W * C), jnp.float32),
            jax.ShapeDtypeStruct((F * H2, W * C), jnp.int32),
        ],
        compiler_params=pltpu.CompilerParams(
            dimension_semantics=("parallel",),
            vmem_limit_bytes=64 * 1024 * 1024),
    )(*args)
    return out, idx


def _patches(xpad, H, W, C):
    nb = xpad.shape[0]
    M = nb * H * W
    cols = [xpad[:, ki:ki + H, kj:kj + W, :].reshape(M, C)
            for ki in range(3) for kj in range(3)]
    return jnp.concatenate(cols, axis=-1)


def _conv_block_body_nhwc(x_ref, *refs, nb, H, W, cins, couts, T):
    ncv = len(cins)
    w_refs = [refs[2 * i] for i in range(ncv)]
    b_refs = [refs[2 * i + 1] for i in range(ncv)]
    out_ref = refs[2 * ncv]
    idx_ref = refs[2 * ncv + 1]

    cur = x_ref[...]
    for i in range(ncv):
        p = _patches(cur, H, W, cins[i])
        y = jnp.dot(p, w_refs[i][...], preferred_element_type=jnp.float32)
        y = jnp.maximum(y + b_refs[i][...], 0.0)
        if i < ncv - 1:
            yr = y.reshape(nb, H, W, couts[i])
            cur = jnp.pad(yr, ((0, 0), (1, 1), (1, 1), (0, 0)))

    C = couts[-1]
    H2, W2 = H // 2, W // 2
    yw = y.reshape(nb * H * W2, 2, C)
    ve, vo = yw[:, 0, :], yw[:, 1, :]
    mw = jnp.maximum(ve, vo)
    djr = jnp.where(ve >= vo, 0, 1).astype(jnp.int32)
    mh = mw.reshape(nb * H2, 2, W2, C)
    jh = djr.reshape(nb * H2, 2, W2, C)
    me, mo = mh[:, 0], mh[:, 1]
    sel = me >= mo
    m = jnp.where(sel, me, mo).reshape(nb, H2, W2, C)
    di = jnp.where(sel, 0, 1).astype(jnp.int32).reshape(nb, H2, W2, C)
    dj = jnp.where(sel, jh[:, 0], jh[:, 1]).reshape(nb, H2, W2, C)

    j = lax.broadcasted_iota(jnp.int32, (nb, H2, W2, C), 0)
    ho = lax.broadcasted_iota(jnp.int32, (nb, H2, W2, C), 1)
    wo = lax.broadcasted_iota(jnp.int32, (nb, H2, W2, C), 2)
    n = pl.program_id(0) * nb + j
    t = n % T
    idx = t * (H * W) + (2 * ho + di) * W + (2 * wo + dj)

    out_ref[...] = jnp.pad(m, ((0, 0), (1, 1), (1, 1), (0, 0)))
    idx_ref[...] = idx


def _conv_block_nhwc(xpad, ws, bs, *, nb, H, T, cins, couts):
    F = xpad.shape[0]
    W = H
    C = couts[-1]
    H2 = H // 2
    in_specs = [pl.BlockSpec((nb, H + 2, W + 2, cins[0]),
                             lambda i: (i, 0, 0, 0))]
    for w, b in zip(ws, bs):
        in_specs.append(pl.BlockSpec(w.shape, lambda i: (0, 0)))
        in_specs.append(pl.BlockSpec(b.shape, lambda i: (0, 0)))
    args = [xpad]
    for w, b in zip(ws, bs):
        args.extend([w, b])
    out, idx = pl.pallas_call(
        functools.partial(_conv_block_body_nhwc, nb=nb, H=H, W=W,
                          cins=cins, couts=couts, T=T),
        grid=(F // nb,),
        in_specs=in_specs,
        out_specs=[
            pl.BlockSpec((nb, H2 + 2, H2 + 2, C), lambda i: (i, 0, 0, 0)),
            pl.BlockSpec((nb, H2, H2, C), lambda i: (i, 0, 0, 0)),
        ],
        out_shape=[
            jax.ShapeDtypeStruct((F, H2 + 2, H2 + 2, C), jnp.float32),
            jax.ShapeDtypeStruct((F, H2, H2, C), jnp.int32),
        ],
        compiler_params=pltpu.CompilerParams(
            dimension_semantics=("parallel",),
            vmem_limit_bytes=64 * 1024 * 1024),
    )(*args)
    return out, idx


def _prep_conv_nhwc(w, b, gamma, beta, rmean, rvar):
    s = gamma / jnp.sqrt(rvar + _BN_EPS)
    wmat = (w * s[:, None, None, None]).transpose(2, 3, 1, 0)
    wmat = wmat.reshape(-1, w.shape[0])
    beff = (b * s + beta - rmean * s).reshape(1, -1)
    return wmat, beff


def _patches4(x, C):
    Bsz = x.shape[0]
    cols = [x[:, ki:ki + 4, kj:kj + 4, :].reshape(Bsz * 16, C)
            for ki in range(3) for kj in range(3)]
    return jnp.concatenate(cols, axis=-1)


def _lstm_layer_body(x_ref, w_ref, b_ref, pci_ref, pcf_ref, pco_ref, *rest,
                     cin, cout, Bsz, T, last):
    if last:
        (wmu_ref, bmu_ref, wlv_ref, blv_ref, eps_ref,
         z_ref, mu_ref, lv_ref, hpad_ref, c_ref) = rest
    else:
        out_ref, hpad_ref, c_ref = rest

    t = pl.program_id(0)

    @pl.when(t == 0)
    def _():
        hpad_ref[...] = jnp.zeros_like(hpad_ref)
        c_ref[...] = jnp.zeros_like(c_ref)

    px = _patches4(x_ref[0], cin)
    ph = _patches4(hpad_ref[...], cout)
    p = jnp.concatenate([px, ph], axis=-1)
    gates = jnp.dot(p, w_ref[...], preferred_element_type=jnp.float32)
    gates = gates + b_ref[...]

    c_prev = c_ref[...]
    i_g = _sigmoid(gates[:, 0 * cout:1 * cout] + pci_ref[...] * c_prev)
    f_g = _sigmoid(gates[:, 1 * cout:2 * cout] + pcf_ref[...] * c_prev)
    c_new = f_g * c_prev + i_g * jnp.tanh(gates[:, 2 * cout:3 * cout])
    o_g = _sigmoid(gates[:, 3 * cout:4 * cout] + pco_ref[...] * c_new)
    h = o_g * jnp.tanh(c_new)

    c_ref[...] = c_new
    hp = jnp.pad(h.reshape(Bsz, 4, 4, cout), ((0, 0), (1, 1), (1, 1), (0, 0)))
    hpad_ref[...] = hp

    if not last:
        out_ref[0] = hp
    else:
        @pl.when(t == T - 1)
        def _():
            flat = h.reshape(Bsz, 16 * cout)
            mu = jnp.dot(flat, wmu_ref[...],
                         preferred_element_type=jnp.float32) + bmu_ref[...]
            lv = jnp.dot(flat, wlv_ref[...],
                         preferred_element_type=jnp.float32) + blv_ref[...]
            z_ref[...] = mu + eps_ref[...] * jnp.exp(0.5 * lv)
            mu_ref[...] = mu
            lv_ref[...] = lv


def _lstm_layer(xseq, wcat, b2, pci, pcf, pco, *, cin, cout, Bsz, T,
                head=None):
    last = head is not None
    K = 9 * (cin + cout)
    in_specs = [
        pl.BlockSpec((1, Bsz, 6, 6, cin), lambda t: (t, 0, 0, 0, 0)),
        pl.BlockSpec((K, 4 * cout), lambda t: (0, 0)),
        pl.BlockSpec((1, 4 * cout), lambda t: (0, 0)),
        pl.BlockSpec((Bsz * 16, cout), lambda t: (0, 0)),
        pl.BlockSpec((Bsz * 16, cout), lambda t: (0, 0)),
        pl.BlockSpec((Bsz * 16, cout), lambda t: (0, 0)),
    ]
    args = [xseq, wcat, b2, pci, pcf, pco]
    if last:
        wmu, bmu, wlv, blv, eps = head
        L = wmu.shape[1]
        in_specs += [
            pl.BlockSpec((16 * cout, L), lambda t: (0, 0)),
            pl.BlockSpec((1, L), lambda t: (0, 0)),
            pl.BlockSpec((16 * cout, L), lambda t: (0, 0)),
            pl.BlockSpec((1, L), lambda t: (0, 0)),
            pl.BlockSpec((Bsz, L), lambda t: (0, 0)),
        ]
        args += [wmu, bmu, wlv, blv, eps]
        out_specs = [pl.BlockSpec((Bsz, L), lambda t: (0, 0))] * 3
        out_shape = [jax.ShapeDtypeStruct((Bsz, L), jnp.float32)] * 3
    else:
        out_specs = [pl.BlockSpec((1, Bsz, 6, 6, cout),
                                  lambda t: (t, 0, 0, 0, 0))]
        out_shape = [jax.ShapeDtypeStruct((T, Bsz, 6, 6, cout), jnp.float32)]
    res = pl.pallas_call(
        functools.partial(_lstm_layer_body, cin=cin, cout=cout, Bsz=Bsz,
                          T=T, last=last),
        grid=(T,),
        in_specs=in_specs,
        out_specs=out_specs,
        out_shape=out_shape,
        scratch_shapes=[
            pltpu.VMEM((Bsz, 6, 6, cout), jnp.float32),
            pltpu.VMEM((Bsz * 16, cout), jnp.float32),
        ],
        compiler_params=pltpu.CompilerParams(
            dimension_semantics=("arbitrary",),
            vmem_limit_bytes=64 * 1024 * 1024),
    )(*args)
    return res


def _prep_conv(w, b, gamma, beta, rmean, rvar, Win, Wout, stride):
    s = gamma / jnp.sqrt(rvar + _BN_EPS)
    cout, cin = w.shape[0], w.shape[1]
    ws = w * s[:, None, None, None]
    wi_idx = jnp.arange(Win)[:, None]
    wo_idx = jnp.arange(Wout)[None, :]
    parts = []
    for ki in range(3):
        acc = jnp.zeros((Win * cin, Wout * cout), jnp.float32)
        for kj in range(3):
            a = (wi_idx == stride * (wo_idx + kj - 1)).astype(jnp.float32)
            wt = ws[:, :, ki, kj].T
            acc = acc + (a[:, None, :, None] * wt[None, :, None, :]
                         ).reshape(Win * cin, Wout * cout)
        parts.append(acc)
    wcat = jnp.concatenate(parts, axis=0)
    beff = (b * s + beta - rmean * s).reshape(1, cout)
    bpacked = jnp.tile(beff, (1, Wout))
    return wcat, bpacked


def _prep_lstm(w, b, cin, cout):
    wx = w[:, :cin].transpose(2, 3, 1, 0).reshape(9 * cin, 4 * cout)
    wh = w[:, cin:].transpose(2, 3, 1, 0).reshape(9 * cout, 4 * cout)
    return jnp.concatenate([wx, wh], axis=0), b.reshape(1, 4 * cout)


def _prep_peep(p, Bsz):
    cout = p.shape[0]
    return jnp.tile(p.transpose(1, 2, 0).reshape(16, cout), (Bsz, 1))


def kernel(x, eps, conv0_w, conv0_b, conv0_gamma, conv0_beta, conv0_rmean, conv0_rvar, conv1_w, conv1_b, conv1_gamma, conv1_beta, conv1_rmean, conv1_rvar, conv2_w, conv2_b, conv2_gamma, conv2_beta, conv2_rmean, conv2_rvar, conv3_w, conv3_b, conv3_gamma, conv3_beta, conv3_rmean, conv3_rvar, conv4_w, conv4_b, conv4_gamma, conv4_beta, conv4_rmean, conv4_rvar, conv5_w, conv5_b, conv5_gamma, conv5_beta, conv5_rmean, conv5_rvar, conv6_w, conv6_b, conv6_gamma, conv6_beta, conv6_rmean, conv6_rvar, conv7_w, conv7_b, conv7_gamma, conv7_beta, conv7_rmean, conv7_rvar, conv8_w, conv8_b, conv8_gamma, conv8_beta, conv8_rmean, conv8_rvar, conv9_w, conv9_b, conv9_gamma, conv9_beta, conv9_rmean, conv9_rvar, lstm0_w, lstm0_b, lstm0_w_ci, lstm0_w_cf, lstm0_w_co, lstm1_w, lstm1_b, lstm1_w_ci, lstm1_w_cf, lstm1_w_co, lstm2_w, lstm2_b, lstm2_w_ci, lstm2_w_cf, lstm2_w_co, fc_mu_w, fc_mu_b, fc_lv_w, fc_lv_b):
    B, cond, T, G, _ = x.shape
    conv_params = [
        (conv0_w, conv0_b, conv0_gamma, conv0_beta, conv0_rmean, conv0_rvar),
        (conv1_w, conv1_b, conv1_gamma, conv1_beta, conv1_rmean, conv1_rvar),
        (conv2_w, conv2_b, conv2_gamma, conv2_beta, conv2_rmean, conv2_rvar),
        (conv3_w, conv3_b, conv3_gamma, conv3_beta, conv3_rmean, conv3_rvar),
        (conv4_w, conv4_b, conv4_gamma, conv4_beta, conv4_rmean, conv4_rvar),
        (conv5_w, conv5_b, conv5_gamma, conv5_beta, conv5_rmean, conv5_rvar),
        (conv6_w, conv6_b, conv6_gamma, conv6_beta, conv6_rmean, conv6_rvar),
        (conv7_w, conv7_b, conv7_gamma, conv7_beta, conv7_rmean, conv7_rvar),
        (conv8_w, conv8_b, conv8_gamma, conv8_beta, conv8_rmean, conv8_rvar),
        (conv9_w, conv9_b, conv9_gamma, conv9_beta, conv9_rmean, conv9_rvar),
    ]
    F = B * T
    cur = x.transpose(0, 2, 3, 4, 1).reshape(F, G, G * cond)

    pool_idx = []
    prev_w = G
    for bi, (lids, H, nb) in enumerate([([0, 1], G, min(16, F))]):
        ws, bs, couts = [], [], []
        for k, i in enumerate(lids):
            stride = 2 if (bi > 0 and k == 0) else 1
            win = prev_w if k == 0 else H
            wcat, bp = _prep_conv(*conv_params[i], Win=win, Wout=H,
                                  stride=stride)
            ws.append(wcat)
            bs.append(bp)
            couts.append(conv_params[i][0].shape[0])
        cur, idx = _conv_block(cur, ws, bs, nb=nb, H=H, T=T, couts=couts)
        H2 = H // 2
        C = couts[-1]
        pool_idx.append(idx.reshape(B, T, H2, H, C)[:, :, :, ::2, :]
                        .transpose(0, 4, 1, 2, 3))
        prev_w = H
        cur = cur.reshape(F, H2, H * C)

    C1 = conv_params[1][0].shape[0]
    Hb = G // 2
    cur = cur.reshape(F, Hb, 2 * Hb, C1)[:, :, ::2, :]
    cur = jnp.pad(cur, ((0, 0), (1, 1), (1, 1), (0, 0)))

    for lids, H, nb in [([2, 3], G // 2, min(4, F)),
                        ([4, 5, 6], G // 4, min(8, F)),
                        ([7, 8, 9], G // 8, min(16, F))]:
        cins = [conv_params[i][0].shape[1] for i in lids]
        couts = [conv_params[i][0].shape[0] for i in lids]
        ws, bs = [], []
        for i in lids:
            wm, be = _prep_conv_nhwc(*conv_params[i])
            ws.append(wm)
            bs.append(be)
        cur, idx = _conv_block_nhwc(cur, ws, bs, nb=nb, H=H, T=T,
                                    cins=cins, couts=couts)
        H2 = H // 2
        C = couts[-1]
        pool_idx.append(idx.reshape(B, T, H2, H2, C)
                        .transpose(0, 4, 1, 2, 3))

    C4 = conv_params[9][0].shape[0]
    y = (cur[:, 1:5, 1:5, :].reshape(B, T, 4, 4, C4)
         .transpose(0, 4, 1, 2, 3))

    xseq = cur.reshape(B, T, 6, 6, C4).transpose(1, 0, 2, 3, 4)
    lstm_params = [
        (lstm0_w, lstm0_b, lstm0_w_ci, lstm0_w_cf, lstm0_w_co),
        (lstm1_w, lstm1_b, lstm1_w_ci, lstm1_w_cf, lstm1_w_co),
        (lstm2_w, lstm2_b, lstm2_w_ci, lstm2_w_cf, lstm2_w_co),
    ]
    L = fc_mu_w.shape[1]
    for li, (w, b, pci, pcf, pco) in enumerate(lstm_params):
        cout = w.shape[0] // 4
        cin = w.shape[1] - cout
        wcat, b2 = _prep_lstm(w, b, cin, cout)
        peeps = [_prep_peep(p, B) for p in (pci, pcf, pco)]
        if li < 2:
            (xseq,) = _lstm_layer(xseq, wcat, b2, *peeps,
                                  cin=cin, cout=cout, Bsz=B, T=T)
        else:
            wmu = (fc_mu_w.reshape(cout, 16, L).transpose(1, 0, 2)
                   .reshape(16 * cout, L))
            wlv = (fc_lv_w.reshape(cout, 16, L).transpose(1, 0, 2)
                   .reshape(16 * cout, L))
            head = (wmu, fc_mu_b.reshape(1, L), wlv, fc_lv_b.reshape(1, L),
                    eps)
            z, mu, logvar = _lstm_layer(xseq, wcat, b2, *peeps,
                                        cin=cin, cout=cout, Bsz=B, T=T,
                                        head=head)
    return z, mu, logvar, pool_idx, y


# speedup vs baseline: 1.0937x; 1.0937x over previous
import functools

import jax
import jax.numpy as jnp
from jax import lax
from jax.experimental import pallas as pl
from jax.experimental.pallas import tpu as pltpu

_BN_EPS = 1e-5


def _sigmoid(x):
    return 1.0 / (1.0 + jnp.exp(-x))


def _conv_block_body(x_ref, *refs, nb, H, T, couts, W):
    ncv = len(couts)
    w_refs = [refs[2 * i] for i in range(ncv)]
    b_refs = [refs[2 * i + 1] for i in range(ncv)]
    m_ref = refs[2 * ncv]
    idx_ref = refs[2 * ncv + 1]

    cur = x_ref[...]
    M = nb * H
    for i in range(ncv):
        hp = jnp.pad(cur, ((0, 0), (1, 1), (0, 0)))
        lhs = jnp.concatenate([hp[:, ki:ki + H, :] for ki in range(3)],
                              axis=-1).reshape(M, -1)
        y = jnp.dot(lhs, w_refs[i][...], preferred_element_type=jnp.float32)
        y = jnp.maximum(y + b_refs[i][...], 0.0)
        if i < ncv - 1:
            cur = y.reshape(nb, H, y.shape[-1])

    C = couts[-1]
    H2 = H // 2
    WC = W * C
    ysh = jnp.pad(y, ((0, 0), (0, C)))[:, C:]
    mw = jnp.maximum(y, ysh)
    djr = jnp.where(y >= ysh, 0, 1).astype(jnp.int32)
    mh = mw.reshape(nb * H2, 2, WC)
    jh = djr.reshape(nb * H2, 2, WC)
    me, mo = mh[:, 0], mh[:, 1]
    sel = me >= mo
    m = jnp.where(sel, me, mo)
    di = jnp.where(sel, 0, 1).astype(jnp.int32)
    dj = jnp.where(sel, jh[:, 0], jh[:, 1])

    row = lax.broadcasted_iota(jnp.int32, (nb * H2, WC), 0)
    lane = lax.broadcasted_iota(jnp.int32, (nb * H2, WC), 1)
    n = pl.program_id(0) * nb + row // H2
    t = n % T
    ho = row % H2
    w = lane // C
    idx = t * (H * W) + (2 * ho + di) * W + (w + dj)

    m_ref[...] = m
    idx_ref[...] = idx


def _conv_block(xp, ws, bs, *, nb, H, T, couts):
    F = xp.shape[0]
    W = H
    C = couts[-1]
    H2 = H // 2
    in_specs = [pl.BlockSpec((nb, H, xp.shape[2]), lambda i: (i, 0, 0))]
    for w, b in zip(ws, bs):
        in_specs.append(pl.BlockSpec(w.shape, lambda i: (0, 0)))
        in_specs.append(pl.BlockSpec(b.shape, lambda i: (0, 0)))
    args = [xp]
    for w, b in zip(ws, bs):
        args.extend([w, b])
    out, idx = pl.pallas_call(
        functools.partial(_conv_block_body, nb=nb, H=H, T=T,
                          couts=couts, W=W),
        grid=(F // nb,),
        in_specs=in_specs,
        out_specs=[
            pl.BlockSpec((nb * H2, W * C), lambda i: (i, 0)),
            pl.BlockSpec((nb * H2, W * C), lambda i: (i, 0)),
        ],
        out_shape=[
            jax.ShapeDtypeStruct((F * H2, W * C), jnp.float32),
            jax.ShapeDtypeStruct((F * H2, W * C), jnp.int32),
        ],
        compiler_params=pltpu.CompilerParams(
            dimension_semantics=("parallel",),
            vmem_limit_bytes=64 * 1024 * 1024),
    )(*args)
    return out, idx


def _patches(xpad, H, W, C):
    nb = xpad.shape[0]
    M = nb * H * W
    cols = [xpad[:, ki:ki + H, kj:kj + W, :].reshape(M, C)
            for ki in range(3) for kj in range(3)]
    return jnp.concatenate(cols, axis=-1)


def _conv_block_body_nhwc(x_ref, *refs, nb, H, W, cins, couts, T):
    ncv = len(cins)
    w_refs = [refs[2 * i] for i in range(ncv)]
    b_refs = [refs[2 * i + 1] for i in range(ncv)]
    out_ref = refs[2 * ncv]
    idx_ref = refs[2 * ncv + 1]

    cur = x_ref[...]
    for i in range(ncv):
        p = _patches(cur, H, W, cins[i])
        y = jnp.dot(p, w_refs[i][...], preferred_element_type=jnp.float32)
        y = jnp.maximum(y + b_refs[i][...], 0.0)
        if i < ncv - 1:
            yr = y.reshape(nb, H, W, couts[i])
            cur = jnp.pad(yr, ((0, 0), (1, 1), (1, 1), (0, 0)))

    C = couts[-1]
    H2, W2 = H // 2, W // 2
    yw = y.reshape(nb * H * W2, 2, C)
    ve, vo = yw[:, 0, :], yw[:, 1, :]
    mw = jnp.maximum(ve, vo)
    djr = jnp.where(ve >= vo, 0, 1).astype(jnp.int32)
    mh = mw.reshape(nb * H2, 2, W2, C)
    jh = djr.reshape(nb * H2, 2, W2, C)
    me, mo = mh[:, 0], mh[:, 1]
    sel = me >= mo
    m = jnp.where(sel, me, mo).reshape(nb, H2, W2, C)
    di = jnp.where(sel, 0, 1).astype(jnp.int32).reshape(nb, H2, W2, C)
    dj = jnp.where(sel, jh[:, 0], jh[:, 1]).reshape(nb, H2, W2, C)

    j = lax.broadcasted_iota(jnp.int32, (nb, H2, W2, C), 0)
    ho = lax.broadcasted_iota(jnp.int32, (nb, H2, W2, C), 1)
    wo = lax.broadcasted_iota(jnp.int32, (nb, H2, W2, C), 2)
    n = pl.program_id(0) * nb + j
    t = n % T
    idx = t * (H * W) + (2 * ho + di) * W + (2 * wo + dj)

    out_ref[...] = jnp.pad(m, ((0, 0), (1, 1), (1, 1), (0, 0)))
    idx_ref[...] = idx


def _conv_block_nhwc(xpad, ws, bs, *, nb, H, T, cins, couts):
    F = xpad.shape[0]
    W = H
    C = couts[-1]
    H2 = H // 2
    in_specs = [pl.BlockSpec((nb, H + 2, W + 2, cins[0]),
                             lambda i: (i, 0, 0, 0))]
    for w, b in zip(ws, bs):
        in_specs.append(pl.BlockSpec(w.shape, lambda i: (0, 0)))
        in_specs.append(pl.BlockSpec(b.shape, lambda i: (0, 0)))
    args = [xpad]
    for w, b in zip(ws, bs):
        args.extend([w, b])
    out, idx = pl.pallas_call(
        functools.partial(_conv_block_body_nhwc, nb=nb, H=H, W=W,
                          cins=cins, couts=couts, T=T),
        grid=(F // nb,),
        in_specs=in_specs,
        out_specs=[
            pl.BlockSpec((nb, H2 + 2, H2 + 2, C), lambda i: (i, 0, 0, 0)),
            pl.BlockSpec((nb, H2, H2, C), lambda i: (i, 0, 0, 0)),
        ],
        out_shape=[
            jax.ShapeDtypeStruct((F, H2 + 2, H2 + 2, C), jnp.float32),
            jax.ShapeDtypeStruct((F, H2, H2, C), jnp.int32),
        ],
        compiler_params=pltpu.CompilerParams(
            dimension_semantics=("parallel",),
            vmem_limit_bytes=64 * 1024 * 1024),
    )(*args)
    return out, idx


def _prep_conv_nhwc(w, b, gamma, beta, rmean, rvar):
    s = gamma / jnp.sqrt(rvar + _BN_EPS)
    wmat = (w * s[:, None, None, None]).transpose(2, 3, 1, 0)
    wmat = wmat.reshape(-1, w.shape[0])
    beff = (b * s + beta - rmean * s).reshape(1, -1)
    return wmat, beff


def _patches4(x, C):
    Bsz = x.shape[0]
    cols = [x[:, ki:ki + 4, kj:kj + 4, :].reshape(Bsz * 16, C)
            for ki in range(3) for kj in range(3)]
    return jnp.concatenate(cols, axis=-1)


def _lstm_layer_body(x_ref, w_ref, b_ref, pci_ref, pcf_ref, pco_ref, *rest,
                     cin, cout, Bsz, T, last):
    if last:
        (wmu_ref, bmu_ref, wlv_ref, blv_ref, eps_ref,
         z_ref, mu_ref, lv_ref, hpad_ref, c_ref) = rest
    else:
        out_ref, hpad_ref, c_ref = rest

    t = pl.program_id(0)

    @pl.when(t == 0)
    def _():
        hpad_ref[...] = jnp.zeros_like(hpad_ref)
        c_ref[...] = jnp.zeros_like(c_ref)

    px = _patches4(x_ref[0], cin)
    ph = _patches4(hpad_ref[...], cout)
    p = jnp.concatenate([px, ph], axis=-1).astype(jnp.bfloat16)
    gates = jnp.dot(p, w_ref[...], preferred_element_type=jnp.float32)
    gates = gates + b_ref[...]

    c_prev = c_ref[...]
    i_g = _sigmoid(gates[:, 0 * cout:1 * cout] + pci_ref[...] * c_prev)
    f_g = _sigmoid(gates[:, 1 * cout:2 * cout] + pcf_ref[...] * c_prev)
    c_new = f_g * c_prev + i_g * jnp.tanh(gates[:, 2 * cout:3 * cout])
    o_g = _sigmoid(gates[:, 3 * cout:4 * cout] + pco_ref[...] * c_new)
    h = o_g * jnp.tanh(c_new)

    c_ref[...] = c_new
    hp = jnp.pad(h.reshape(Bsz, 4, 4, cout), ((0, 0), (1, 1), (1, 1), (0, 0)))
    hpad_ref[...] = hp

    if not last:
        out_ref[0] = hp
    else:
        @pl.when(t == T - 1)
        def _():
            flat = h.reshape(Bsz, 16 * cout)
            mu = jnp.dot(flat, wmu_ref[...],
                         preferred_element_type=jnp.float32) + bmu_ref[...]
            lv = jnp.dot(flat, wlv_ref[...],
                         preferred_element_type=jnp.float32) + blv_ref[...]
            z_ref[...] = mu + eps_ref[...] * jnp.exp(0.5 * lv)
            mu_ref[...] = mu
            lv_ref[...] = lv


def _lstm_layer(xseq, wcat, b2, pci, pcf, pco, *, cin, cout, Bsz, T,
                head=None):
    last = head is not None
    K = 9 * (cin + cout)
    in_specs = [
        pl.BlockSpec((1, Bsz, 6, 6, cin), lambda t: (t, 0, 0, 0, 0)),
        pl.BlockSpec((K, 4 * cout), lambda t: (0, 0)),
        pl.BlockSpec((1, 4 * cout), lambda t: (0, 0)),
        pl.BlockSpec((Bsz * 16, cout), lambda t: (0, 0)),
        pl.BlockSpec((Bsz * 16, cout), lambda t: (0, 0)),
        pl.BlockSpec((Bsz * 16, cout), lambda t: (0, 0)),
    ]
    args = [xseq, wcat, b2, pci, pcf, pco]
    if last:
        wmu, bmu, wlv, blv, eps = head
        L = wmu.shape[1]
        in_specs += [
            pl.BlockSpec((16 * cout, L), lambda t: (0, 0)),
            pl.BlockSpec((1, L), lambda t: (0, 0)),
            pl.BlockSpec((16 * cout, L), lambda t: (0, 0)),
            pl.BlockSpec((1, L), lambda t: (0, 0)),
            pl.BlockSpec((Bsz, L), lambda t: (0, 0)),
        ]
        args += [wmu, bmu, wlv, blv, eps]
        out_specs = [pl.BlockSpec((Bsz, L), lambda t: (0, 0))] * 3
        out_shape = [jax.ShapeDtypeStruct((Bsz, L), jnp.float32)] * 3
    else:
        out_specs = [pl.BlockSpec((1, Bsz, 6, 6, cout),
                                  lambda t: (t, 0, 0, 0, 0))]
        out_shape = [jax.ShapeDtypeStruct((T, Bsz, 6, 6, cout), jnp.float32)]
    res = pl.pallas_call(
        functools.partial(_lstm_layer_body, cin=cin, cout=cout, Bsz=Bsz,
                          T=T, last=last),
        grid=(T,),
        in_specs=in_specs,
        out_specs=out_specs,
        out_shape=out_shape,
        scratch_shapes=[
            pltpu.VMEM((Bsz, 6, 6, cout), jnp.float32),
            pltpu.VMEM((Bsz * 16, cout), jnp.float32),
        ],
        compiler_params=pltpu.CompilerParams(
            dimension_semantics=("arbitrary",),
            vmem_limit_bytes=64 * 1024 * 1024),
    )(*args)
    return res


def _prep_conv(w, b, gamma, beta, rmean, rvar, Win, Wout, stride):
    s = gamma / jnp.sqrt(rvar + _BN_EPS)
    cout, cin = w.shape[0], w.shape[1]
    ws = w * s[:, None, None, None]
    wi_idx = jnp.arange(Win)[:, None]
    wo_idx = jnp.arange(Wout)[None, :]
    parts = []
    for ki in range(3):
        acc = jnp.zeros((Win * cin, Wout * cout), jnp.float32)
        for kj in range(3):
            a = (wi_idx == stride * (wo_idx + kj - 1)).astype(jnp.float32)
            wt = ws[:, :, ki, kj].T
            acc = acc + (a[:, None, :, None] * wt[None, :, None, :]
                         ).reshape(Win * cin, Wout * cout)
        parts.append(acc)
    wcat = jnp.concatenate(parts, axis=0)
    beff = (b * s + beta - rmean * s).reshape(1, cout)
    bpacked = jnp.tile(beff, (1, Wout))
    return wcat, bpacked


def _prep_lstm(w, b, cin, cout):
    wx = w[:, :cin].transpose(2, 3, 1, 0).reshape(9 * cin, 4 * cout)
    wh = w[:, cin:].transpose(2, 3, 1, 0).reshape(9 * cout, 4 * cout)
    wcat = jnp.concatenate([wx, wh], axis=0).astype(jnp.bfloat16)
    return wcat, b.reshape(1, 4 * cout)


def _prep_peep(p, Bsz):
    cout = p.shape[0]
    return jnp.tile(p.transpose(1, 2, 0).reshape(16, cout), (Bsz, 1))


def kernel(x, eps, conv0_w, conv0_b, conv0_gamma, conv0_beta, conv0_rmean, conv0_rvar, conv1_w, conv1_b, conv1_gamma, conv1_beta, conv1_rmean, conv1_rvar, conv2_w, conv2_b, conv2_gamma, conv2_beta, conv2_rmean, conv2_rvar, conv3_w, conv3_b, conv3_gamma, conv3_beta, conv3_rmean, conv3_rvar, conv4_w, conv4_b, conv4_gamma, conv4_beta, conv4_rmean, conv4_rvar, conv5_w, conv5_b, conv5_gamma, conv5_beta, conv5_rmean, conv5_rvar, conv6_w, conv6_b, conv6_gamma, conv6_beta, conv6_rmean, conv6_rvar, conv7_w, conv7_b, conv7_gamma, conv7_beta, conv7_rmean, conv7_rvar, conv8_w, conv8_b, conv8_gamma, conv8_beta, conv8_rmean, conv8_rvar, conv9_w, conv9_b, conv9_gamma, conv9_beta, conv9_rmean, conv9_rvar, lstm0_w, lstm0_b, lstm0_w_ci, lstm0_w_cf, lstm0_w_co, lstm1_w, lstm1_b, lstm1_w_ci, lstm1_w_cf, lstm1_w_co, lstm2_w, lstm2_b, lstm2_w_ci, lstm2_w_cf, lstm2_w_co, fc_mu_w, fc_mu_b, fc_lv_w, fc_lv_b):
    B, cond, T, G, _ = x.shape
    conv_params = [
        (conv0_w, conv0_b, conv0_gamma, conv0_beta, conv0_rmean, conv0_rvar),
        (conv1_w, conv1_b, conv1_gamma, conv1_beta, conv1_rmean, conv1_rvar),
        (conv2_w, conv2_b, conv2_gamma, conv2_beta, conv2_rmean, conv2_rvar),
        (conv3_w, conv3_b, conv3_gamma, conv3_beta, conv3_rmean, conv3_rvar),
        (conv4_w, conv4_b, conv4_gamma, conv4_beta, conv4_rmean, conv4_rvar),
        (conv5_w, conv5_b, conv5_gamma, conv5_beta, conv5_rmean, conv5_rvar),
        (conv6_w, conv6_b, conv6_gamma, conv6_beta, conv6_rmean, conv6_rvar),
        (conv7_w, conv7_b, conv7_gamma, conv7_beta, conv7_rmean, conv7_rvar),
        (conv8_w, conv8_b, conv8_gamma, conv8_beta, conv8_rmean, conv8_rvar),
        (conv9_w, conv9_b, conv9_gamma, conv9_beta, conv9_rmean, conv9_rvar),
    ]
    F = B * T
    cur = x.transpose(0, 2, 3, 4, 1).reshape(F, G, G * cond)

    pool_idx = []
    prev_w = G
    for bi, (lids, H, nb) in enumerate(
            [([0, 1], G, min(16, F)), ([2, 3], G // 2, min(8, F))]):
        ws, bs, couts = [], [], []
        for k, i in enumerate(lids):
            stride = 2 if (bi > 0 and k == 0) else 1
            win = prev_w if k == 0 else H
            wcat, bp = _prep_conv(*conv_params[i], Win=win, Wout=H,
                                  stride=stride)
            ws.append(wcat)
            bs.append(bp)
            couts.append(conv_params[i][0].shape[0])
        cur, idx = _conv_block(cur, ws, bs, nb=nb, H=H, T=T, couts=couts)
        H2 = H // 2
        C = couts[-1]
        pool_idx.append(idx.reshape(B, T, H2, H, C)[:, :, :, ::2, :]
                        .transpose(0, 4, 1, 2, 3))
        prev_w = H
        cur = cur.reshape(F, H2, H * C)

    C2 = conv_params[3][0].shape[0]
    Hb = G // 4
    cur = cur.reshape(F, Hb, 2 * Hb, C2)[:, :, ::2, :]
    cur = jnp.pad(cur, ((0, 0), (1, 1), (1, 1), (0, 0)))

    for lids, H, nb in [([4, 5, 6], G // 4, min(8, F)),
                        ([7, 8, 9], G // 8, min(16, F))]:
        cins = [conv_params[i][0].shape[1] for i in lids]
        couts = [conv_params[i][0].shape[0] for i in lids]
        ws, bs = [], []
        for i in lids:
            wm, be = _prep_conv_nhwc(*conv_params[i])
            ws.append(wm)
            bs.append(be)
        cur, idx = _conv_block_nhwc(cur, ws, bs, nb=nb, H=H, T=T,
                                    cins=cins, couts=couts)
        H2 = H // 2
        C = couts[-1]
        pool_idx.append(idx.reshape(B, T, H2, H2, C)
                        .transpose(0, 4, 1, 2, 3))

    C4 = conv_params[9][0].shape[0]
    y = (cur[:, 1:5, 1:5, :].reshape(B, T, 4, 4, C4)
         .transpose(0, 4, 1, 2, 3))

    xseq = cur.reshape(B, T, 6, 6, C4).transpose(1, 0, 2, 3, 4)
    lstm_params = [
        (lstm0_w, lstm0_b, lstm0_w_ci, lstm0_w_cf, lstm0_w_co),
        (lstm1_w, lstm1_b, lstm1_w_ci, lstm1_w_cf, lstm1_w_co),
        (lstm2_w, lstm2_b, lstm2_w_ci, lstm2_w_cf, lstm2_w_co),
    ]
    L = fc_mu_w.shape[1]
    for li, (w, b, pci, pcf, pco) in enumerate(lstm_params):
        cout = w.shape[0] // 4
        cin = w.shape[1] - cout
        wcat, b2 = _prep_lstm(w, b, cin, cout)
        peeps = [_prep_peep(p, B) for p in (pci, pcf, pco)]
        if li < 2:
            (xseq,) = _lstm_layer(xseq, wcat, b2, *peeps,
                                  cin=cin, cout=cout, Bsz=B, T=T)
        else:
            wmu = (fc_mu_w.reshape(cout, 16, L).transpose(1, 0, 2)
                   .reshape(16 * cout, L))
            wlv = (fc_lv_w.reshape(cout, 16, L).transpose(1, 0, 2)
                   .reshape(16 * cout, L))
            head = (wmu, fc_mu_b.reshape(1, L), wlv, fc_lv_b.reshape(1, L),
                    eps)
            z, mu, logvar = _lstm_layer(xseq, wcat, b2, *peeps,
                                        cin=cin, cout=cout, Bsz=B, T=T,
                                        head=head)
    return z, mu, logvar, pool_idx, y
